# baseline (speedup 1.0000x reference)
import dataclasses
import functools
import math

import jax
import jax.numpy as jnp
from jax import lax
from jax.experimental import pallas as pl
from jax.experimental.pallas import tpu as pltpu
from jax.experimental.pallas import tpu_sc as plsc

NC = 2
NS = 16
NWORK = NC * NS
W = 64


def _acc_rows(n):
    return -(-(n + 8) // W) * W


_MESH = plsc.VectorSubcoreMesh(core_axis_name="c", subcore_axis_name="s")


def _sc_segment_sum(table, srcp, dstp, with_count=False):
    N, D = table.shape
    wpw = srcp.shape[1]
    assert wpw % 2 == 0
    hw = 16 if with_count else wpw // 2
    assert hw % 8 == 0 and hw % 2 == 0 and wpw % hw == 0
    Npad = _acc_rows(N)
    NBLK = Npad // W
    KMAX = -(-NBLK // NS)

    out_type = [jax.ShapeDtypeStruct((NC, Npad, D), jnp.float32)]
    scratch = [
        pltpu.VMEM((hw, W), jnp.int32),
        pltpu.VMEM((hw, W), jnp.int32),
        pltpu.VMEM((W, D), jnp.float32),
        pltpu.VMEM((W, D), jnp.float32),
        pltpu.VMEM_SHARED((Npad, D), jnp.float32),
        pltpu.SemaphoreType.DMA,
        pltpu.SemaphoreType.DMA,
        pltpu.SemaphoreType.DMA,
        pltpu.SemaphoreType.DMA,
    ]
    if with_count:
        out_type.append(jax.ShapeDtypeStruct((NWORK, Npad), jnp.float32))
        scratch.append(pltpu.VMEM((Npad,), jnp.float32))

    cp = pltpu.CompilerParams()
    if "needs_layout_passes" in pltpu.CompilerParams.__dataclass_fields__:
        cp = dataclasses.replace(cp, needs_layout_passes=False)

    @functools.partial(pl.kernel, mesh=_MESH, out_type=out_type,
                       compiler_params=cp, scratch_types=scratch)
    def run(table_h, src_h, dst_h, *refs):
        if with_count:
            (out_h, cnt_h, sidx, didx, rows0, rows1, agg_sh,
             gsem0, gsem1, ssem0, ssem1, hist) = refs
        else:
            (out_h, sidx, didx, rows0, rows1, agg_sh,
             gsem0, gsem1, ssem0, ssem1) = refs
        cid = lax.axis_index("c")
        sid = lax.axis_index("s")
        wid = sid * NC + cid

        zv = jnp.zeros((16,), jnp.float32)

        @pl.loop(0, W)
        def _(r):
            @pl.loop(0, D, step=16)
            def _(c2):
                rows0[r, pl.ds(c2, 16)] = zv

        for k in range(KMAX):
            bid = sid + k * NS

            @pl.when(bid < NBLK)
            def _():
                pltpu.sync_copy(rows0, agg_sh.at[pl.ds(bid * W, W)])

        if with_count:
            zv = jnp.zeros((16,), jnp.float32)
            onev = jnp.ones((16,), jnp.float32)

            @pl.loop(0, Npad, step=16)
            def _(r):
                hist[pl.ds(r, 16)] = zv

        plsc.subcore_barrier()

        for h in range(wpw // hw):
            pltpu.sync_copy(src_h.at[wid, pl.ds(h * hw, hw)], sidx)
            pltpu.sync_copy(dst_h.at[wid, pl.ds(h * hw, hw)], didx)
            half = hw // 2
            pltpu.async_copy(table_h.at[sidx.at[0]], rows0, gsem0)

            @pl.loop(0, half)
            def _(k):
                j0 = 2 * k
                pltpu.make_async_copy(table_h.at[sidx.at[j0]], rows0,
                                      gsem0).wait()
                pltpu.async_copy(rows0, agg_sh.at[didx.at[j0]], ssem0,
                                 add=True)

                @pl.when(k > 0)
                def _():
                    pltpu.make_async_copy(rows1, agg_sh.at[didx.at[j0 + 1]],
                                          ssem1).wait()

                pltpu.async_copy(table_h.at[sidx.at[j0 + 1]], rows1, gsem1)

                if with_count:
                    onev_ = jnp.ones((16,), jnp.float32)
                    for wj in range(2):
                        for k4 in range(W // 16):
                            iv = didx[j0 + wj, pl.ds(k4 * 16, 16)]
                            plsc.addupdate_scatter(hist, [iv], onev_)

                pltpu.make_async_copy(table_h.at[sidx.at[j0 + 1]], rows1,
                                      gsem1).wait()
                pltpu.async_copy(rows1, agg_sh.at[didx.at[j0 + 1]], ssem1,
                                 add=True)

                pltpu.make_async_copy(rows0, agg_sh.at[didx.at[j0]],
                                      ssem0).wait()

                @pl.when(k < half - 1)
                def _():
                    pltpu.async_copy(table_h.at[sidx.at[j0 + 2]], rows0,
                                     gsem0)

            pltpu.make_async_copy(rows1, agg_sh.at[didx.at[0]],
                                  ssem1).wait()
        plsc.subcore_barrier()

        for k in range(KMAX):
            bid = sid + k * NS

            @pl.when(bid < NBLK)
            def _():
                pltpu.sync_copy(agg_sh.at[pl.ds(bid * W, W)],
                                out_h.at[cid, pl.ds(bid * W, W)])

        if with_count:
            pltpu.sync_copy(hist, cnt_h.at[wid])

    res = run(table, srcp, dstp)
    return tuple(res) if with_count else res[0]


def _xw(xin, wT, b):
    N, D = xin.shape
    BLK = 2048

    def body(x_ref, w_ref, b_ref, o_ref):
        o_ref[...] = jnp.dot(x_ref[...], w_ref[...],
                             preferred_element_type=jnp.float32,
                             precision=lax.Precision.HIGHEST) + b_ref[...]

    return pl.pallas_call(
        body,
        grid=(pl.cdiv(N, BLK),),
        in_specs=[pl.BlockSpec((BLK, D), lambda i: (i, 0)),
                  pl.BlockSpec((D, D), lambda i: (0, 0)),
                  pl.BlockSpec((1, D), lambda i: (0, 0))],
        out_specs=pl.BlockSpec((BLK, D), lambda i: (i, 0)),
        out_shape=jax.ShapeDtypeStruct((N, D), jnp.float32),
    )(xin, wT, b)


def _dense_stage(parts, cnts, xr, wlT, gamma, beta, apply_bn):
    N, D = xr.shape
    BLK = 2048
    grid = (pl.cdiv(N, BLK),)

    def body(p_ref, c_ref, xr_ref, wl_ref, *rest):
        if apply_bn:
            g_ref, b_ref, o_ref = rest
        else:
            (o_ref,) = rest
        agg = p_ref[0] + p_ref[1]
        cnt = lax.dot_general(c_ref[...], jnp.ones((NWORK, 1), jnp.float32),
                              (((0,), (0,)), ((), ())),
                              preferred_element_type=jnp.float32,
                              precision=lax.Precision.HIGHEST)
        mean = agg / jnp.maximum(cnt, 1.0)
        acc = jnp.dot(mean, wl_ref[...], preferred_element_type=jnp.float32,
                      precision=lax.Precision.HIGHEST)
        acc = acc + xr_ref[...]
        nrm = jnp.sqrt(jnp.sum(acc * acc, axis=1, keepdims=True))
        acc = acc / jnp.maximum(nrm, 1e-12)
        if apply_bn:
            acc = jnp.maximum(acc, 0.0)
            acc = g_ref[...] * acc * (1.0 / math.sqrt(1.0 + 1e-5)) + b_ref[...]
        o_ref[...] = acc

    in_specs = [
        pl.BlockSpec((NC, BLK, D), lambda i: (0, i, 0)),
        pl.BlockSpec((NWORK, BLK), lambda i: (0, i)),
        pl.BlockSpec((BLK, D), lambda i: (i, 0)),
        pl.BlockSpec((D, D), lambda i: (0, 0)),
    ]
    args = [parts, cnts, xr, wlT]
    if apply_bn:
        in_specs += [pl.BlockSpec((1, D), lambda i: (0, 0)),
                     pl.BlockSpec((1, D), lambda i: (0, 0))]
        args += [gamma, beta]

    return pl.pallas_call(
        body,
        grid=grid,
        in_specs=in_specs,
        out_specs=pl.BlockSpec((BLK, D), lambda i: (i, 0)),
        out_shape=jax.ShapeDtypeStruct((N, D), jnp.float32),
    )(*args)


def kernel(x, edge_index, W1l, b1l, W1r, bn_gamma, bn_beta, W2l, b2l, W2r):
    N, D = x.shape
    E = edge_index.shape[1]
    src = edge_index[0]
    dst = edge_index[1]

    wpw = -(-(-(-E // (NWORK * W))) // 16) * 16
    Ep = NWORK * W * wpw
    pad = Ep - E
    ndummy = _acc_rows(N) - N
    ar = jnp.arange(pad, dtype=jnp.int32)
    pad_src = ar % N
    pad_dst = N + (ar % ndummy)
    if E % NWORK == 0 and pad % NWORK == 0:
        srcp = jnp.concatenate(
            [src.reshape(NWORK, -1), pad_src.reshape(NWORK, -1)], axis=1)
        dstp = jnp.concatenate(
            [dst.reshape(NWORK, -1), pad_dst.reshape(NWORK, -1)], axis=1)
    else:
        srcp = jnp.concatenate([src, pad_src]).reshape(NWORK, -1)
        dstp = jnp.concatenate([dst, pad_dst]).reshape(NWORK, -1)
    srcp = srcp.reshape(NWORK, wpw, W)
    dstp = dstp.reshape(NWORK, wpw, W)

    xr1 = _xw(x, W1r.T, b1l.reshape(1, -1))
    parts1, cnts = _sc_segment_sum(x, srcp, dstp, with_count=True)
    h = _dense_stage(parts1, cnts, xr1, W1l.T,
                     bn_gamma.reshape(1, -1), bn_beta.reshape(1, -1), True)
    xr2 = _xw(h, W2r.T, b2l.reshape(1, -1))
    parts2 = _sc_segment_sum(h, srcp, dstp)
    out = _dense_stage(parts2, cnts, xr2, W2l.T, None, None, False)
    return out


# speedup vs baseline: 1.3445x; 1.3445x over previous
import dataclasses
import functools
import math

import jax
import jax.numpy as jnp
from jax import lax
from jax.experimental import pallas as pl
from jax.experimental.pallas import tpu as pltpu
from jax.experimental.pallas import tpu_sc as plsc

NC = 2
NS = 16
NWORK = NC * NS
W = 64
NSLOT = 4


def _acc_rows(n):
    return -(-(n + 8) // W) * W


_MESH = plsc.VectorSubcoreMesh(core_axis_name="c", subcore_axis_name="s")


def _sc_segment_sum(table, srcp, dstp, with_count=False):
    N, D = table.shape
    wpw = srcp.shape[1]
    assert wpw % 2 == 0
    hw = 16 if with_count else 40
    assert hw % 8 == 0 and hw % NSLOT == 0 and wpw % hw == 0
    Npad = _acc_rows(N)
    NBLK = Npad // W
    KMAX = -(-NBLK // NS)

    out_type = [jax.ShapeDtypeStruct((NC, Npad, D), jnp.float32)]
    scratch = (
        [pltpu.VMEM((hw, W), jnp.int32),
         pltpu.VMEM((hw, W), jnp.int32)]
        + [pltpu.VMEM((W, D), jnp.float32) for _ in range(NSLOT)]
        + [pltpu.VMEM_SHARED((Npad, D), jnp.float32)]
        + [pltpu.SemaphoreType.DMA for _ in range(2 * NSLOT)]
    )
    if with_count:
        out_type.append(jax.ShapeDtypeStruct((NWORK, Npad), jnp.float32))
        scratch.append(pltpu.VMEM((Npad,), jnp.float32))

    cp = pltpu.CompilerParams()
    if "needs_layout_passes" in pltpu.CompilerParams.__dataclass_fields__:
        cp = dataclasses.replace(cp, needs_layout_passes=False)

    @functools.partial(pl.kernel, mesh=_MESH, out_type=out_type,
                       compiler_params=cp, scratch_types=scratch)
    def run(table_h, src_h, dst_h, *refs):
        if with_count:
            out_h, cnt_h = refs[0], refs[1]
            refs = refs[2:]
            hist = refs[3 + 3 * NSLOT]
        else:
            out_h = refs[0]
            refs = refs[1:]
        sidx, didx = refs[0], refs[1]
        rows = refs[2:2 + NSLOT]
        agg_sh = refs[2 + NSLOT]
        gsem = refs[3 + NSLOT:3 + 2 * NSLOT]
        ssem = refs[3 + 2 * NSLOT:3 + 3 * NSLOT]
        cid = lax.axis_index("c")
        sid = lax.axis_index("s")
        wid = sid * NC + cid

        zv = jnp.zeros((16,), jnp.float32)
        rows0 = rows[0]

        @pl.loop(0, W)
        def _(r):
            @pl.loop(0, D, step=16)
            def _(c2):
                rows0[r, pl.ds(c2, 16)] = zv

        for k in range(KMAX):
            bid = sid + k * NS

            @pl.when(bid < NBLK)
            def _():
                pltpu.sync_copy(rows0, agg_sh.at[pl.ds(bid * W, W)])

        if with_count:
            zv = jnp.zeros((16,), jnp.float32)
            onev = jnp.ones((16,), jnp.float32)

            @pl.loop(0, Npad, step=16)
            def _(r):
                hist[pl.ds(r, 16)] = zv

        plsc.subcore_barrier()

        for h in range(wpw // hw):
            pltpu.sync_copy(src_h.at[wid, pl.ds(h * hw, hw)], sidx)
            pltpu.sync_copy(dst_h.at[wid, pl.ds(h * hw, hw)], didx)

            @pl.loop(0, hw // NSLOT)
            def _(q):
                j0 = NSLOT * q
                for s in range(NSLOT):
                    @pl.when(q > 0)
                    def _():
                        pltpu.make_async_copy(
                            rows[s], agg_sh.at[didx.at[j0 + s]],
                            ssem[s]).wait()

                    pltpu.async_copy(table_h.at[sidx.at[j0 + s]], rows[s],
                                     gsem[s])

                if with_count:
                    onev_ = jnp.ones((16,), jnp.float32)
                    for wj in range(NSLOT):
                        for k4 in range(W // 16):
                            iv = didx[j0 + wj, pl.ds(k4 * 16, 16)]
                            plsc.addupdate_scatter(hist, [iv], onev_)

                for s in range(NSLOT):
                    pltpu.make_async_copy(table_h.at[sidx.at[j0 + s]],
                                          rows[s], gsem[s]).wait()
                    pltpu.async_copy(rows[s], agg_sh.at[didx.at[j0 + s]],
                                     ssem[s], add=True)

            for s in range(NSLOT):
                pltpu.make_async_copy(rows[s], agg_sh.at[didx.at[s]],
                                      ssem[s]).wait()
        plsc.subcore_barrier()

        for k in range(KMAX):
            bid = sid + k * NS

            @pl.when(bid < NBLK)
            def _():
                pltpu.sync_copy(agg_sh.at[pl.ds(bid * W, W)],
                                out_h.at[cid, pl.ds(bid * W, W)])

        if with_count:
            pltpu.sync_copy(hist, cnt_h.at[wid])

    res = run(table, srcp, dstp)
    return tuple(res) if with_count else res[0]


def _xw(xin, wT, b):
    N, D = xin.shape
    BLK = 2048

    def body(x_ref, w_ref, b_ref, o_ref):
        o_ref[...] = jnp.dot(x_ref[...], w_ref[...],
                             preferred_element_type=jnp.float32,
                             precision=lax.Precision.HIGHEST) + b_ref[...]

    return pl.pallas_call(
        body,
        grid=(pl.cdiv(N, BLK),),
        in_specs=[pl.BlockSpec((BLK, D), lambda i: (i, 0)),
                  pl.BlockSpec((D, D), lambda i: (0, 0)),
                  pl.BlockSpec((1, D), lambda i: (0, 0))],
        out_specs=pl.BlockSpec((BLK, D), lambda i: (i, 0)),
        out_shape=jax.ShapeDtypeStruct((N, D), jnp.float32),
    )(xin, wT, b)


def _dense_stage(parts, cnts, xr, wlT, gamma, beta, apply_bn):
    N, D = xr.shape
    BLK = 2048
    grid = (pl.cdiv(N, BLK),)

    def body(p_ref, c_ref, xr_ref, wl_ref, *rest):
        if apply_bn:
            g_ref, b_ref, o_ref = rest
        else:
            (o_ref,) = rest
        agg = p_ref[0] + p_ref[1]
        cnt = lax.dot_general(c_ref[...], jnp.ones((NWORK, 1), jnp.float32),
                              (((0,), (0,)), ((), ())),
                              preferred_element_type=jnp.float32,
                              precision=lax.Precision.HIGHEST)
        mean = agg / jnp.maximum(cnt, 1.0)
        acc = jnp.dot(mean, wl_ref[...], preferred_element_type=jnp.float32,
                      precision=lax.Precision.HIGHEST)
        acc = acc + xr_ref[...]
        nrm = jnp.sqrt(jnp.sum(acc * acc, axis=1, keepdims=True))
        acc = acc / jnp.maximum(nrm, 1e-12)
        if apply_bn:
            acc = jnp.maximum(acc, 0.0)
            acc = g_ref[...] * acc * (1.0 / math.sqrt(1.0 + 1e-5)) + b_ref[...]
        o_ref[...] = acc

    in_specs = [
        pl.BlockSpec((NC, BLK, D), lambda i: (0, i, 0)),
        pl.BlockSpec((NWORK, BLK), lambda i: (0, i)),
        pl.BlockSpec((BLK, D), lambda i: (i, 0)),
        pl.BlockSpec((D, D), lambda i: (0, 0)),
    ]
    args = [parts, cnts, xr, wlT]
    if apply_bn:
        in_specs += [pl.BlockSpec((1, D), lambda i: (0, 0)),
                     pl.BlockSpec((1, D), lambda i: (0, 0))]
        args += [gamma, beta]

    return pl.pallas_call(
        body,
        grid=grid,
        in_specs=in_specs,
        out_specs=pl.BlockSpec((BLK, D), lambda i: (i, 0)),
        out_shape=jax.ShapeDtypeStruct((N, D), jnp.float32),
    )(*args)


def kernel(x, edge_index, W1l, b1l, W1r, bn_gamma, bn_beta, W2l, b2l, W2r):
    N, D = x.shape
    E = edge_index.shape[1]
    src = edge_index[0]
    dst = edge_index[1]

    wpw = -(-(-(-E // (NWORK * W))) // 16) * 16
    Ep = NWORK * W * wpw
    pad = Ep - E
    ndummy = _acc_rows(N) - N
    ar = jnp.arange(pad, dtype=jnp.int32)
    pad_src = ar % N
    pad_dst = N + (ar % ndummy)
    if E % NWORK == 0 and pad % NWORK == 0:
        srcp = jnp.concatenate(
            [src.reshape(NWORK, -1), pad_src.reshape(NWORK, -1)], axis=1)
        dstp = jnp.concatenate(
            [dst.reshape(NWORK, -1), pad_dst.reshape(NWORK, -1)], axis=1)
    else:
        srcp = jnp.concatenate([src, pad_src]).reshape(NWORK, -1)
        dstp = jnp.concatenate([dst, pad_dst]).reshape(NWORK, -1)
    srcp = srcp.reshape(NWORK, wpw, W)
    dstp = dstp.reshape(NWORK, wpw, W)

    xr1 = _xw(x, W1r.T, b1l.reshape(1, -1))
    parts1, cnts = _sc_segment_sum(x, srcp, dstp, with_count=True)
    h = _dense_stage(parts1, cnts, xr1, W1l.T,
                     bn_gamma.reshape(1, -1), bn_beta.reshape(1, -1), True)
    xr2 = _xw(h, W2r.T, b2l.reshape(1, -1))
    parts2 = _sc_segment_sum(h, srcp, dstp)
    out = _dense_stage(parts2, cnts, xr2, W2l.T, None, None, False)
    return out
